# baseline (speedup 1.0000x reference)
import jax
import jax.numpy as jnp
import numpy as np
from jax import lax
from jax.experimental import pallas as pl
from jax.experimental.pallas import tpu as pltpu
from jax.experimental.pallas import tpu_sc as plsc

_CATS = 26
_D = 32
_BATCH = 16384
_TOTAL = _BATCH * _CATS

_NC, _NS = 2, 16
_NW = _NC * _NS
_PER_W = _TOTAL // _NW

_CHUNK = 1024
_N_CHUNKS = _PER_W // _CHUNK
_IDX_PER_DMA = 128
_DMAS_PER_CHUNK = _CHUNK // _IDX_PER_DMA


def _sc_body(x_hbm, offs_hbm, table_hbm, out_hbm, x_v, offs_v, rows_v, sem):
    wid = lax.axis_index("s") * _NC + lax.axis_index("c")
    base = wid * _PER_W

    pltpu.sync_copy(x_hbm.at[pl.ds(base, _PER_W)], x_v)
    pltpu.sync_copy(offs_hbm, offs_v)

    def add_body(i, _):
        sl = pl.ds(i * 16, 16)
        x_v[sl] = x_v[sl] + offs_v[sl]
        return 0

    lax.fori_loop(0, _PER_W // 16, add_body, 0, unroll=8)

    def chunk_body(k, _):
        start = k * _CHUNK
        for j in range(_DMAS_PER_CHUNK):
            idx_sl = x_v.at[pl.ds(start + j * _IDX_PER_DMA, _IDX_PER_DMA)]
            dst = rows_v.at[pl.ds(j * _IDX_PER_DMA, _IDX_PER_DMA)]
            pltpu.async_copy(table_hbm.at[idx_sl], dst, sem)
        for j in range(_DMAS_PER_CHUNK):
            idx_sl = x_v.at[pl.ds(start + j * _IDX_PER_DMA, _IDX_PER_DMA)]
            dst = rows_v.at[pl.ds(j * _IDX_PER_DMA, _IDX_PER_DMA)]
            pltpu.make_async_copy(table_hbm.at[idx_sl], dst, sem).wait()
        pltpu.sync_copy(rows_v, out_hbm.at[pl.ds(base + start, _CHUNK)])
        return 0

    lax.fori_loop(0, _N_CHUNKS, chunk_body, 0)


@jax.jit
def _run(x_flat, offs_tiled, emb_weight):
    k = pl.kernel(
        _sc_body,
        out_type=jax.ShapeDtypeStruct((_TOTAL, _D), jnp.float32),
        mesh=plsc.VectorSubcoreMesh(core_axis_name="c", subcore_axis_name="s",
                                    num_cores=_NC, num_subcores=_NS),
        scratch_types=[
            pltpu.VMEM((_PER_W,), jnp.int32),
            pltpu.VMEM((_PER_W,), jnp.int32),
            pltpu.VMEM((_CHUNK, _D), jnp.float32),
            pltpu.SemaphoreType.DMA,
        ],
        compiler_params=pltpu.CompilerParams(use_tc_tiling_on_sc=False),
    )
    return k(x_flat, offs_tiled, emb_weight)


def kernel(x_cat, emb_weight):
    offsets = np.cumsum([0] + [100000] * (_CATS - 1)).astype(np.int32)
    offs_tiled = jnp.asarray(np.tile(offsets, _PER_W // _CATS))
    x_flat = x_cat.reshape(-1)
    out = _run(x_flat, offs_tiled, emb_weight)
    return out.reshape(_BATCH, _CATS, _D)
